# baseline (speedup 1.0000x reference)
import functools

import jax
import jax.numpy as jnp
from jax import lax
from jax.experimental import pallas as pl
from jax.experimental.pallas import tpu as pltpu
from jax.experimental.pallas import tpu_sc as plsc

_N_TOTAL = 100000
_N_REL = 16
_ENT_DIM = 128
_REL_DIM = 64
_B = 4096
_LAM = 1e-05

_NUM_WORKERS = 32
_HALF = _B // 2
_CHUNK = _HALF // _NUM_WORKERS
_HROWS = 3 * _HALF

_BLK = 2048
_WCOLS = _N_REL * _REL_DIM


def _sc_gather(table, idx):
    mesh = plsc.VectorSubcoreMesh(core_axis_name="c", subcore_axis_name="s")

    @functools.partial(
        pl.kernel,
        mesh=mesh,
        out_type=jax.ShapeDtypeStruct((_HROWS, _ENT_DIM), jnp.float32),
        scratch_types=[
            pltpu.VMEM((3 * _CHUNK,), jnp.int32),
            pltpu.VMEM((3 * _CHUNK, _ENT_DIM), jnp.float32),
            pltpu.SemaphoreType.DMA,
            pltpu.SemaphoreType.DMA,
            pltpu.SemaphoreType.DMA,
            pltpu.SemaphoreType.DMA,
        ],
    )
    def gather_kernel(table_hbm, idx_hbm, out_hbm,
                      idx_v, rows_v, s0, s1, s2, sw):
        wid = lax.axis_index("s") * 2 + lax.axis_index("c")
        base = wid * _CHUNK
        pltpu.sync_copy(idx_hbm.at[pl.ds(wid * 3 * _CHUNK, 3 * _CHUNK)],
                        idx_v)
        sems = (s0, s1, s2)
        gathers = [
            pltpu.async_copy(
                table_hbm.at[idx_v.at[pl.ds(j * _CHUNK, _CHUNK)]],
                rows_v.at[pl.ds(j * _CHUNK, _CHUNK)],
                sems[j],
            )
            for j in range(3)
        ]
        writes = []
        for j in range(3):
            gathers[j].wait()
            writes.append(
                pltpu.async_copy(
                    rows_v.at[pl.ds(j * _CHUNK, _CHUNK)],
                    out_hbm.at[pl.ds(j * _HALF + base, _CHUNK)],
                    sw,
                )
            )
        for w in writes:
            w.wait()

    return gather_kernel(table, idx)


def _tc_body(h_ref, p_ref, n_ref, r_ref, w_ref, rel_ref, s_ref, out_ref):
    w = w_ref[...]
    hb = h_ref[...].astype(jnp.bfloat16)
    pb = p_ref[...].astype(jnp.bfloat16)
    nb = n_ref[...].astype(jnp.bfloat16)
    ah = jnp.dot(hb, w, preferred_element_type=jnp.float32).astype(jnp.bfloat16)
    ap = jnp.dot(pb, w, preferred_element_type=jnp.float32).astype(jnp.bfloat16)
    an = jnp.dot(nb, w, preferred_element_type=jnp.float32).astype(jnp.bfloat16)
    r_all = rel_ref[...]
    s = s_ref[...]

    s2 = ah + r_all
    d1 = an - ap
    s3 = (s2 + s2) - (ap + an)
    diff16 = jnp.dot(d1 * s3, s,
                     preferred_element_type=jnp.float32)
    t16 = jnp.dot(ah * ah + ap * ap + an * an, s,
                  preferred_element_type=jnp.float32)
    e16 = jnp.dot(r_all * r_all, s,
                  preferred_element_type=jnp.float32)

    rv = r_ref[...]
    cols = lax.broadcasted_iota(jnp.int32, (_BLK, _N_REL), 1)
    m = (rv == cols).astype(jnp.float32)

    x = jnp.sum(m * diff16, axis=1, keepdims=True)
    sp = jnp.maximum(x, 0.0) + jnp.log1p(jnp.exp(-jnp.abs(x)))
    sql2 = jnp.sum(m * (t16 + e16), axis=(0, 1), keepdims=True)
    out_ref[...] = (jnp.sum(sp, axis=(0, 1), keepdims=True)[:, :1]
                    + (_LAM * 0.5) * sql2)


def _tc_loss(gathered, r2, w_all, rel_flat, s_mat):
    return pl.pallas_call(
        _tc_body,
        grid=(1,),
        in_specs=[
            pl.BlockSpec((_BLK, _ENT_DIM), lambda g: (0, 0)),
            pl.BlockSpec((_BLK, _ENT_DIM), lambda g: (1, 0)),
            pl.BlockSpec((_BLK, _ENT_DIM), lambda g: (2, 0)),
            pl.BlockSpec((_BLK, 1), lambda g: (0, 0)),
            pl.BlockSpec((_ENT_DIM, _WCOLS), lambda g: (0, 0)),
            pl.BlockSpec((1, _WCOLS), lambda g: (0, 0)),
            pl.BlockSpec((_WCOLS, _N_REL), lambda g: (0, 0)),
        ],
        out_specs=pl.BlockSpec((1, 1), lambda g: (0, 0)),
        out_shape=jax.ShapeDtypeStruct((1, 1), jnp.float32),
    )(gathered, gathered, gathered, r2, w_all, rel_flat, s_mat)


def _interleave_idx(h, p, n):
    return jnp.stack(
        [h.reshape(_NUM_WORKERS, _CHUNK),
         p.reshape(_NUM_WORKERS, _CHUNK),
         n.reshape(_NUM_WORKERS, _CHUNK)], axis=1).reshape(-1)


def kernel(h, r, pos_t, neg_t, relation_embed, entity_user_embed, W_R):
    idx_a = _interleave_idx(h[:_HALF], pos_t[:_HALF], neg_t[:_HALF])
    idx_b = _interleave_idx(h[_HALF:], pos_t[_HALF:], neg_t[_HALF:])
    gath_a = _sc_gather(entity_user_embed, idx_a)
    gath_b = _sc_gather(entity_user_embed, idx_b)
    w_all = jnp.transpose(W_R, (1, 0, 2)).reshape(_ENT_DIM, _WCOLS
                                                  ).astype(jnp.bfloat16)
    rel_flat = relation_embed.reshape(1, _WCOLS).astype(jnp.bfloat16)
    col_ids = jnp.arange(_WCOLS, dtype=jnp.int32) // _REL_DIM
    s_mat = (col_ids[:, None] == jnp.arange(_N_REL, dtype=jnp.int32)[None, :]
             ).astype(jnp.bfloat16)
    ra = r[:_HALF].reshape(_HALF, 1)
    rb = r[_HALF:].reshape(_HALF, 1)
    out_a = _tc_loss(gath_a, ra, w_all, rel_flat, s_mat)
    out_b = _tc_loss(gath_b, rb, w_all, rel_flat, s_mat)
    return (out_a[0, 0] + out_b[0, 0]) * (1.0 / _B)


# speedup vs baseline: 1.1214x; 1.1214x over previous
import functools

import jax
import jax.numpy as jnp
from jax import lax
from jax.experimental import pallas as pl
from jax.experimental.pallas import tpu as pltpu
from jax.experimental.pallas import tpu_sc as plsc

_N_TOTAL = 100000
_N_REL = 16
_ENT_DIM = 128
_REL_DIM = 64
_B = 4096
_LAM = 1e-05

_GATHER_ROWS = 3 * _B
_NUM_WORKERS = 32
_CHUNK = _B // _NUM_WORKERS

_BLK = 2048
_NB = _B // _BLK
_WCOLS = _N_REL * _REL_DIM


def _sc_gather(table, idx):
    mesh = plsc.VectorSubcoreMesh(core_axis_name="c", subcore_axis_name="s")

    @functools.partial(
        pl.kernel,
        mesh=mesh,
        out_type=jax.ShapeDtypeStruct((_GATHER_ROWS, _ENT_DIM), jnp.float32),
        scratch_types=[
            pltpu.VMEM((3 * _CHUNK,), jnp.int32),
            pltpu.VMEM((3 * _CHUNK, _ENT_DIM), jnp.float32),
            pltpu.SemaphoreType.DMA,
        ],
    )
    def gather_kernel(table_hbm, idx_hbm, out_hbm, idx_v, rows_v, sem):
        wid = lax.axis_index("s") * 2 + lax.axis_index("c")
        base = wid * 3 * _CHUNK
        pltpu.sync_copy(idx_hbm.at[pl.ds(base, 3 * _CHUNK)], idx_v)
        gathers = [
            pltpu.async_copy(
                table_hbm.at[idx_v.at[pl.ds(j * _CHUNK, _CHUNK)]],
                rows_v.at[pl.ds(j * _CHUNK, _CHUNK)],
                sem,
            )
            for j in range(3)
        ]
        for g in gathers:
            g.wait()
        pltpu.sync_copy(rows_v, out_hbm.at[pl.ds(base, 3 * _CHUNK)])

    return gather_kernel(table, idx)


def _tc_body(h_ref, p_ref, n_ref, r_ref, w_ref, rel_ref, s_ref, out_ref):
    g = pl.program_id(0)
    w = w_ref[...]
    hb = h_ref[...].astype(jnp.bfloat16)
    pb = p_ref[...].astype(jnp.bfloat16)
    nb = n_ref[...].astype(jnp.bfloat16)
    ah = jnp.dot(hb, w, preferred_element_type=jnp.float32).astype(jnp.bfloat16)
    ap = jnp.dot(pb, w, preferred_element_type=jnp.float32).astype(jnp.bfloat16)
    an = jnp.dot(nb, w, preferred_element_type=jnp.float32).astype(jnp.bfloat16)
    r_all = rel_ref[...]
    s = s_ref[...]

    s2 = ah + r_all
    d1 = an - ap
    s3 = (s2 + s2) - (ap + an)
    diff16 = jnp.dot(d1 * s3, s,
                     preferred_element_type=jnp.float32)
    t16 = jnp.dot(ah * ah + ap * ap + an * an, s,
                  preferred_element_type=jnp.float32)
    e16 = jnp.dot(r_all * r_all, s,
                  preferred_element_type=jnp.float32)

    rv = r_ref[...]
    cols = lax.broadcasted_iota(jnp.int32, (_BLK, _N_REL), 1)
    m = (rv == cols).astype(jnp.float32)

    x = jnp.sum(m * diff16, axis=1, keepdims=True)
    sp = jnp.maximum(x, 0.0) + jnp.log1p(jnp.exp(-jnp.abs(x)))
    sql2 = jnp.sum(m * (t16 + e16), axis=(0, 1), keepdims=True)
    part = (jnp.sum(sp, axis=(0, 1), keepdims=True)[:, :1]
            + (_LAM * 0.5) * sql2)

    @pl.when(g == 0)
    def _init():
        out_ref[...] = jnp.zeros((1, 1), jnp.float32)

    out_ref[...] = out_ref[...] + part

    @pl.when(g == _NB - 1)
    def _finalize():
        out_ref[...] = out_ref[...] * (1.0 / _B)


def _tc_loss(gathered, r2, w_all, rel_flat, s_mat):
    return pl.pallas_call(
        _tc_body,
        grid=(_NB,),
        in_specs=[
            pl.BlockSpec((_BLK, _ENT_DIM), lambda g: (g, 0)),
            pl.BlockSpec((_BLK, _ENT_DIM), lambda g: (g + _NB, 0)),
            pl.BlockSpec((_BLK, _ENT_DIM), lambda g: (g + 2 * _NB, 0)),
            pl.BlockSpec((_BLK, 1), lambda g: (g, 0)),
            pl.BlockSpec((_ENT_DIM, _WCOLS), lambda g: (0, 0)),
            pl.BlockSpec((1, _WCOLS), lambda g: (0, 0)),
            pl.BlockSpec((_WCOLS, _N_REL), lambda g: (0, 0)),
        ],
        out_specs=pl.BlockSpec((1, 1), lambda g: (0, 0)),
        out_shape=jax.ShapeDtypeStruct((1, 1), jnp.float32),
    )(gathered, gathered, gathered, r2, w_all, rel_flat, s_mat)


def kernel(h, r, pos_t, neg_t, relation_embed, entity_user_embed, W_R):
    idx = jnp.concatenate([h, pos_t, neg_t])
    gathered = _sc_gather(entity_user_embed, idx)
    w_all = jnp.transpose(W_R, (1, 0, 2)).reshape(_ENT_DIM, _WCOLS
                                                  ).astype(jnp.bfloat16)
    rel_flat = relation_embed.reshape(1, _WCOLS).astype(jnp.bfloat16)
    col_ids = jnp.arange(_WCOLS, dtype=jnp.int32) // _REL_DIM
    s_mat = (col_ids[:, None] == jnp.arange(_N_REL, dtype=jnp.int32)[None, :]
             ).astype(jnp.bfloat16)
    r2 = r.reshape(_B, 1)
    out = _tc_loss(gathered, r2, w_all, rel_flat, s_mat)
    return out[0, 0]
